# Initial kernel scaffold; baseline (speedup 1.0000x reference)
import jax
import jax.numpy as jnp
from jax.experimental import pallas as pl


def kernel(edge_index, node_latent, edge_forces, edge_torques, W1m, b1m, W2m, b2m, W1i, b1i, W2i, b2i):
    raise NotImplementedError("write your pallas kernel here")


# speedup vs baseline: 1.3386x; 1.3386x over previous
import functools

import jax
import jax.numpy as jnp
from jax import lax
from jax.experimental import pallas as pl
from jax.experimental.pallas import tpu as pltpu
from jax.experimental.pallas import tpu_sc as plsc

_NC = 2
_NS = 16
_NW = _NC * _NS
_NCOMP = 6
_NSHARD = 5
_CH = 1024
_L = 16


def _sc_scatter(recv, forces, torques, n):
    e = recv.shape[0]
    eps = e // _NSHARD
    nchunk = eps // _CH
    mesh = plsc.VectorSubcoreMesh(core_axis_name="c", subcore_axis_name="s")

    @functools.partial(
        pl.kernel,
        out_type=jax.ShapeDtypeStruct((_NCOMP, _NSHARD, n), jnp.float32),
        mesh=mesh,
        scratch_types=[
            pltpu.VMEM((n,), jnp.float32),
            pltpu.VMEM((_CH,), jnp.int32),
            pltpu.VMEM((_CH, 3), jnp.float32),
        ],
        compiler_params=pltpu.CompilerParams(
            use_tc_tiling_on_sc=False, needs_layout_passes=False),
    )
    def k(recv_h, f_h, t_h, out, acc, ibuf, pbuf):
        c = lax.axis_index("c")
        s = lax.axis_index("s")
        w = s * _NC + c

        @pl.when(w < _NCOMP * _NSHARD)
        def _():
            comp = w // _NSHARD
            shard = w - comp * _NSHARD
            col = lax.rem(comp, 3)

            def zero(i, carry):
                acc[pl.ds(i * _L, _L)] = jnp.zeros((_L,), jnp.float32)
                return carry

            lax.fori_loop(0, n // _L, zero, 0)

            lanes = lax.iota(jnp.int32, _L)
            colv = jnp.broadcast_to(col, (_L,)).astype(jnp.int32)
            base0 = shard * eps

            def chunk(i, carry):
                b = base0 + i * _CH
                pltpu.sync_copy(recv_h.at[pl.ds(b, _CH)], ibuf)

                @pl.when(comp < 3)
                def _():
                    pltpu.sync_copy(f_h.at[pl.ds(b, _CH)], pbuf)

                @pl.when(comp >= 3)
                def _():
                    pltpu.sync_copy(t_h.at[pl.ds(b, _CH)], pbuf)

                def group(j, carry2):
                    r = ibuf[pl.ds(j * _L, _L)]
                    rows = lanes + j * _L
                    v = plsc.load_gather(pbuf, [rows, colv])
                    plsc.addupdate_scatter(acc, [r], v)
                    return carry2

                lax.fori_loop(0, _CH // _L, group, 0, unroll=4)
                return carry

            lax.fori_loop(0, nchunk, chunk, 0)
            pltpu.sync_copy(acc, out.at[comp, shard])

    return k(recv, forces, torques)


def _tc_mlp_combine(x, w1m, b1m, w2m, b2m, w1i, b1i, w2i, b2i, part, bn):
    n, latent = x.shape
    grid = (n // bn,)

    def body(x_ref, w1m_r, b1m_r, w2m_r, b2m_r, w1i_r, b1i_r, w2i_r, b2i_r,
             p_r, dv_r, dw_r):
        xb = x_ref[...]
        hm = jnp.maximum(
            jnp.dot(xb, w1m_r[...], preferred_element_type=jnp.float32)
            + b1m_r[...], 0.0)
        im = jnp.dot(hm, w2m_r[...], preferred_element_type=jnp.float32) + b2m_r[...]
        hi = jnp.maximum(
            jnp.dot(xb, w1i_r[...], preferred_element_type=jnp.float32)
            + b1i_r[...], 0.0)
        ii = jnp.dot(hi, w2i_r[...], preferred_element_type=jnp.float32) + b2i_r[...]
        p = p_r[...]
        net = p[:, 0] + p[:, 1] + p[:, 2] + p[:, 3] + p[:, 4]
        f = jnp.transpose(net[0:3])
        t = jnp.transpose(net[3:6])
        dv_r[...] = im * f
        dw_r[...] = ii * t

    wspec = pl.BlockSpec((latent, latent), lambda i: (0, 0))
    b1spec = pl.BlockSpec((1, latent), lambda i: (0, 0))
    w2spec = pl.BlockSpec((latent, 1), lambda i: (0, 0))
    b2spec = pl.BlockSpec((1, 1), lambda i: (0, 0))
    pspec = pl.BlockSpec((_NCOMP, _NSHARD, bn), lambda i: (0, 0, i))

    return pl.pallas_call(
        body,
        grid=grid,
        in_specs=[
            pl.BlockSpec((bn, latent), lambda i: (i, 0)),
            wspec, b1spec, w2spec, b2spec,
            wspec, b1spec, w2spec, b2spec,
            pspec,
        ],
        out_specs=[
            pl.BlockSpec((bn, 3), lambda i: (i, 0)),
            pl.BlockSpec((bn, 3), lambda i: (i, 0)),
        ],
        out_shape=[
            jax.ShapeDtypeStruct((n, 3), jnp.float32),
            jax.ShapeDtypeStruct((n, 3), jnp.float32),
        ],
    )(x, w1m.reshape(latent, latent), b1m.reshape(1, latent),
      w2m.reshape(latent, 1), b2m.reshape(1, 1),
      w1i.reshape(latent, latent), b1i.reshape(1, latent),
      w2i.reshape(latent, 1), b2i.reshape(1, 1), part)


def kernel(edge_index, node_latent, edge_forces, edge_torques,
           W1m, b1m, W2m, b2m, W1i, b1i, W2i, b2i):
    n = node_latent.shape[0]
    recv = edge_index[1].astype(jnp.int32)
    bn = 2048
    npad = -(-n // bn) * bn
    part = _sc_scatter(recv, edge_forces, edge_torques, npad)
    xp = jnp.pad(node_latent, ((0, npad - n), (0, 0)))
    dv, dw = _tc_mlp_combine(xp, W1m, b1m, W2m, b2m,
                             W1i, b1i, W2i, b2i, part, bn)
    return (dv[:n], dw[:n])


# speedup vs baseline: 7.7298x; 5.7747x over previous
import functools

import jax
import jax.numpy as jnp
from jax import lax
from jax.experimental import pallas as pl
from jax.experimental.pallas import tpu as pltpu
from jax.experimental.pallas import tpu_sc as plsc

_NC = 2
_NS = 16
_NCOMP = 6
_NSHARD = 5
_CH = 1024
_L = 16


def _tc_format(forces, torques, be):
    e = forces.shape[0]
    rows = be // 128
    grid = (e // be,)

    def body(f_r, t_r, fx, fy, fz, tx, ty, tz):
        ft = jnp.transpose(f_r[...])
        tt = jnp.transpose(t_r[...])
        fx[...] = jnp.reshape(ft[0], (rows, 128))
        fy[...] = jnp.reshape(ft[1], (rows, 128))
        fz[...] = jnp.reshape(ft[2], (rows, 128))
        tx[...] = jnp.reshape(tt[0], (rows, 128))
        ty[...] = jnp.reshape(tt[1], (rows, 128))
        tz[...] = jnp.reshape(tt[2], (rows, 128))

    ospec = pl.BlockSpec((rows, 128), lambda i: (i, 0))
    oshape = jax.ShapeDtypeStruct((e // 128, 128), jnp.float32)
    return pl.pallas_call(
        body,
        grid=grid,
        in_specs=[pl.BlockSpec((be, 3), lambda i: (i, 0))] * 2,
        out_specs=[ospec] * 6,
        out_shape=[oshape] * 6,
    )(forces, torques)


def _sc_scatter(recv, comps, n):
    e = recv.shape[0]
    eps = e // _NSHARD
    nchunk = eps // _CH
    crows = _CH // 128
    mesh = plsc.VectorSubcoreMesh(core_axis_name="c", subcore_axis_name="s")

    @functools.partial(
        pl.kernel,
        out_type=jax.ShapeDtypeStruct((_NCOMP, _NSHARD, n), jnp.float32),
        mesh=mesh,
        scratch_types=[
            pltpu.VMEM((n,), jnp.float32),
            pltpu.VMEM((_CH,), jnp.int32),
            pltpu.VMEM((crows, 128), jnp.float32),
        ],
        compiler_params=pltpu.CompilerParams(
            use_tc_tiling_on_sc=False, needs_layout_passes=False),
    )
    def k(recv_h, fx_h, fy_h, fz_h, tx_h, ty_h, tz_h, out, acc, ibuf, pbuf):
        c = lax.axis_index("c")
        s = lax.axis_index("s")
        w = s * _NC + c

        @pl.when(w < _NCOMP * _NSHARD)
        def _():
            comp = w // _NSHARD
            shard = w - comp * _NSHARD

            def zero(i, carry):
                acc[pl.ds(i * _L, _L)] = jnp.zeros((_L,), jnp.float32)
                return carry

            lax.fori_loop(0, n // _L, zero, 0)

            def chunk(i, carry):
                b = shard * eps + i * _CH
                row0 = b // 128
                pltpu.sync_copy(recv_h.at[pl.ds(b, _CH)], ibuf)
                for ci, ch in enumerate((fx_h, fy_h, fz_h, tx_h, ty_h, tz_h)):
                    @pl.when(comp == ci)
                    def _(ch=ch):
                        pltpu.sync_copy(ch.at[pl.ds(row0, crows)], pbuf)

                def rowbody(r8, carry2):
                    for sub in range(128 // _L):
                        r = ibuf[pl.ds(r8 * 128 + sub * _L, _L)]
                        v = pbuf[r8, pl.ds(sub * _L, _L)]
                        plsc.addupdate_scatter(acc, [r], v)
                    return carry2

                lax.fori_loop(0, crows, rowbody, 0)
                return carry

            lax.fori_loop(0, nchunk, chunk, 0)
            pltpu.sync_copy(acc, out.at[comp, shard])

    return k(recv, *comps)


def _tc_mlp_combine(x, w1m, b1m, w2m, b2m, w1i, b1i, w2i, b2i, part, bn):
    n, latent = x.shape
    grid = (n // bn,)

    def body(x_ref, w1m_r, b1m_r, w2m_r, b2m_r, w1i_r, b1i_r, w2i_r, b2i_r,
             p_r, dv_r, dw_r):
        xb = x_ref[...]
        hm = jnp.maximum(
            jnp.dot(xb, w1m_r[...], preferred_element_type=jnp.float32)
            + b1m_r[...], 0.0)
        im = jnp.dot(hm, w2m_r[...], preferred_element_type=jnp.float32) + b2m_r[...]
        hi = jnp.maximum(
            jnp.dot(xb, w1i_r[...], preferred_element_type=jnp.float32)
            + b1i_r[...], 0.0)
        ii = jnp.dot(hi, w2i_r[...], preferred_element_type=jnp.float32) + b2i_r[...]
        p = p_r[...]
        net = p[:, 0] + p[:, 1] + p[:, 2] + p[:, 3] + p[:, 4]
        f = jnp.transpose(net[0:3])
        t = jnp.transpose(net[3:6])
        dv_r[...] = im * f
        dw_r[...] = ii * t

    wspec = pl.BlockSpec((latent, latent), lambda i: (0, 0))
    b1spec = pl.BlockSpec((1, latent), lambda i: (0, 0))
    w2spec = pl.BlockSpec((latent, 1), lambda i: (0, 0))
    b2spec = pl.BlockSpec((1, 1), lambda i: (0, 0))
    pspec = pl.BlockSpec((_NCOMP, _NSHARD, bn), lambda i: (0, 0, i))

    return pl.pallas_call(
        body,
        grid=grid,
        in_specs=[
            pl.BlockSpec((bn, latent), lambda i: (i, 0)),
            wspec, b1spec, w2spec, b2spec,
            wspec, b1spec, w2spec, b2spec,
            pspec,
        ],
        out_specs=[
            pl.BlockSpec((bn, 3), lambda i: (i, 0)),
            pl.BlockSpec((bn, 3), lambda i: (i, 0)),
        ],
        out_shape=[
            jax.ShapeDtypeStruct((n, 3), jnp.float32),
            jax.ShapeDtypeStruct((n, 3), jnp.float32),
        ],
    )(x, w1m.reshape(latent, latent), b1m.reshape(1, latent),
      w2m.reshape(latent, 1), b2m.reshape(1, 1),
      w1i.reshape(latent, latent), b1i.reshape(1, latent),
      w2i.reshape(latent, 1), b2i.reshape(1, 1), part)


def kernel(edge_index, node_latent, edge_forces, edge_torques,
           W1m, b1m, W2m, b2m, W1i, b1i, W2i, b2i):
    n = node_latent.shape[0]
    recv = edge_index[1].astype(jnp.int32)
    comps = _tc_format(edge_forces, edge_torques, 5120)
    bn = 2048
    npad = -(-n // bn) * bn
    part = _sc_scatter(recv, comps, npad)
    xp = jnp.pad(node_latent, ((0, npad - n), (0, 0)))
    dv, dw = _tc_mlp_combine(xp, W1m, b1m, W2m, b2m,
                             W1i, b1i, W2i, b2i, part, bn)
    return (dv[:n], dw[:n])


# speedup vs baseline: 8.8836x; 1.1493x over previous
import functools

import jax
import jax.numpy as jnp
from jax import lax
from jax.experimental import pallas as pl
from jax.experimental.pallas import tpu as pltpu
from jax.experimental.pallas import tpu_sc as plsc

_NC = 2
_NS = 16
_NCOMP = 6
_NSHARD = 5
_CH = 5120
_L = 16


def _tc_format(edge_index, forces, torques, be):
    e = forces.shape[0]
    rows = be // 128
    grid = (e // be,)

    def body(ei_r, f_r, t_r, rv, fx, fy, fz, tx, ty, tz):
        rv[...] = jnp.reshape(ei_r[1], (rows, 128)).astype(jnp.int32)
        ft = jnp.transpose(f_r[...])
        tt = jnp.transpose(t_r[...])
        fx[...] = jnp.reshape(ft[0], (rows, 128))
        fy[...] = jnp.reshape(ft[1], (rows, 128))
        fz[...] = jnp.reshape(ft[2], (rows, 128))
        tx[...] = jnp.reshape(tt[0], (rows, 128))
        ty[...] = jnp.reshape(tt[1], (rows, 128))
        tz[...] = jnp.reshape(tt[2], (rows, 128))

    ospec = pl.BlockSpec((rows, 128), lambda i: (i, 0))
    oshape = jax.ShapeDtypeStruct((e // 128, 128), jnp.float32)
    return pl.pallas_call(
        body,
        grid=grid,
        in_specs=[pl.BlockSpec((2, be), lambda i: (0, i))]
        + [pl.BlockSpec((be, 3), lambda i: (i, 0))] * 2,
        out_specs=[ospec] * 7,
        out_shape=[jax.ShapeDtypeStruct((e // 128, 128), jnp.int32)]
        + [oshape] * 6,
    )(edge_index, forces, torques)


def _sc_scatter(recv2d, comps, n):
    erows = recv2d.shape[0]
    srows = erows // _NSHARD
    crows = _CH // 128
    nchunk = srows // crows
    mesh = plsc.VectorSubcoreMesh(core_axis_name="c", subcore_axis_name="s")

    @functools.partial(
        pl.kernel,
        out_type=jax.ShapeDtypeStruct((_NCOMP, _NSHARD, n), jnp.float32),
        mesh=mesh,
        scratch_types=[
            pltpu.VMEM((n,), jnp.float32),
            pltpu.VMEM((crows, 128), jnp.int32),
            pltpu.VMEM((crows, 128), jnp.float32),
        ],
        compiler_params=pltpu.CompilerParams(
            use_tc_tiling_on_sc=False, needs_layout_passes=False),
    )
    def k(recv_h, fx_h, fy_h, fz_h, tx_h, ty_h, tz_h, out, acc, ibuf, pbuf):
        c = lax.axis_index("c")
        s = lax.axis_index("s")
        w = s * _NC + c

        @pl.when(w < _NCOMP * _NSHARD)
        def _():
            comp = w // _NSHARD
            shard = w - comp * _NSHARD

            def zero(i, carry):
                acc[pl.ds(i * _L, _L)] = jnp.zeros((_L,), jnp.float32)
                return carry

            lax.fori_loop(0, n // _L, zero, 0)

            def chunk(i, carry):
                row0 = shard * srows + i * crows
                pltpu.sync_copy(recv_h.at[pl.ds(row0, crows)], ibuf)
                for ci, ch in enumerate((fx_h, fy_h, fz_h, tx_h, ty_h, tz_h)):
                    @pl.when(comp == ci)
                    def _(ch=ch):
                        pltpu.sync_copy(ch.at[pl.ds(row0, crows)], pbuf)

                def rowbody(r8, carry2):
                    for sub in range(128 // _L):
                        r = ibuf[r8, pl.ds(sub * _L, _L)]
                        v = pbuf[r8, pl.ds(sub * _L, _L)]
                        plsc.addupdate_scatter(acc, [r], v)
                    return carry2

                lax.fori_loop(0, crows, rowbody, 0)
                return carry

            lax.fori_loop(0, nchunk, chunk, 0)
            pltpu.sync_copy(acc, out.at[comp, shard])

    return k(recv2d, *comps)


def _tc_mlp_combine(x, w1m, b1m, w2m, b2m, w1i, b1i, w2i, b2i, part, bn):
    n, latent = x.shape
    grid = (n // bn,)

    def body(x_ref, w1m_r, b1m_r, w2m_r, b2m_r, w1i_r, b1i_r, w2i_r, b2i_r,
             p_r, dv_r, dw_r):
        xb = x_ref[...]
        hm = jnp.maximum(
            jnp.dot(xb, w1m_r[...], preferred_element_type=jnp.float32)
            + b1m_r[...], 0.0)
        im = jnp.dot(hm, w2m_r[...], preferred_element_type=jnp.float32) + b2m_r[...]
        hi = jnp.maximum(
            jnp.dot(xb, w1i_r[...], preferred_element_type=jnp.float32)
            + b1i_r[...], 0.0)
        ii = jnp.dot(hi, w2i_r[...], preferred_element_type=jnp.float32) + b2i_r[...]
        p = p_r[...]
        net = p[:, 0] + p[:, 1] + p[:, 2] + p[:, 3] + p[:, 4]
        f = jnp.transpose(net[0:3])
        t = jnp.transpose(net[3:6])
        dv_r[...] = im * f
        dw_r[...] = ii * t

    wspec = pl.BlockSpec((latent, latent), lambda i: (0, 0))
    b1spec = pl.BlockSpec((1, latent), lambda i: (0, 0))
    w2spec = pl.BlockSpec((latent, 1), lambda i: (0, 0))
    b2spec = pl.BlockSpec((1, 1), lambda i: (0, 0))
    pspec = pl.BlockSpec((_NCOMP, _NSHARD, bn), lambda i: (0, 0, i))

    return pl.pallas_call(
        body,
        grid=grid,
        in_specs=[
            pl.BlockSpec((bn, latent), lambda i: (i, 0)),
            wspec, b1spec, w2spec, b2spec,
            wspec, b1spec, w2spec, b2spec,
            pspec,
        ],
        out_specs=[
            pl.BlockSpec((bn, 3), lambda i: (i, 0)),
            pl.BlockSpec((bn, 3), lambda i: (i, 0)),
        ],
        out_shape=[
            jax.ShapeDtypeStruct((n, 3), jnp.float32),
            jax.ShapeDtypeStruct((n, 3), jnp.float32),
        ],
    )(x, w1m.reshape(latent, latent), b1m.reshape(1, latent),
      w2m.reshape(latent, 1), b2m.reshape(1, 1),
      w1i.reshape(latent, latent), b1i.reshape(1, latent),
      w2i.reshape(latent, 1), b2i.reshape(1, 1), part)


def kernel(edge_index, node_latent, edge_forces, edge_torques,
           W1m, b1m, W2m, b2m, W1i, b1i, W2i, b2i):
    n = node_latent.shape[0]
    recv2d, *comps = _tc_format(edge_index, edge_forces, edge_torques, 5120)
    bn = 2048
    npad = -(-n // bn) * bn
    part = _sc_scatter(recv2d, comps, npad)
    xp = jnp.pad(node_latent, ((0, npad - n), (0, 0)))
    dv, dw = _tc_mlp_combine(xp, W1m, b1m, W2m, b2m,
                             W1i, b1i, W2i, b2i, part, bn)
    return (dv[:n], dw[:n])
